# baseline (speedup 1.0000x reference)
import jax
import jax.numpy as jnp
import numpy as np
from jax.experimental import pallas as pl
from jax.experimental.pallas import tpu as pltpu

B, N, F = 256, 128, 64
H = 256
H2 = 128
L = 64

BB = 32

_EPS = np.asarray(jax.random.normal(jax.random.key(1), (B, L), dtype=jnp.float32))


def _dot(a_, b_):
    return jnp.dot(a_, b_, preferred_element_type=jnp.float32)


def _conv_kernel(x_ref, a_ref, w1_ref, b1_ref, w2_ref, b2_ref,
                 w3_ref, b3_ref, w4_ref, b4_ref, out_ref):
    h = x_ref[...].reshape(BB * N, F)

    def gcn(h2d, w_ref, b_ref):
        m = _dot(h2d, w_ref[...])
        b = b_ref[...]
        outs = []
        for g in range(BB):
            ag = a_ref[g]
            mg = m[g * N:(g + 1) * N]
            outs.append(jax.nn.relu(_dot(ag, mg) + b))
        return jnp.concatenate(outs, axis=0)

    h = gcn(h, w1_ref, b1_ref)
    h = gcn(h, w2_ref, b2_ref)
    h = gcn(h, w3_ref, b3_ref)
    m = _dot(h, w4_ref[...])
    b4 = b4_ref[...]
    for g in range(BB):
        out_ref[:, g, :] = jax.nn.relu(_dot(a_ref[g], m[g * N:(g + 1) * N]) + b4)


def _dense_kernel(h_ref, d1w_ref, d1b_ref, d2w_ref, d2b_ref,
                  dtw_ref, dtb_ref, zmw_ref, zmb_ref, zvw_ref, zvb_ref,
                  eps_ref, zm_ref, zv_ref, z_ref, acc_ref):
    n = pl.program_id(0)
    part = _dot(h_ref[0], d1w_ref[...])

    @pl.when(n == 0)
    def _():
        acc_ref[...] = part + d1b_ref[...]

    @pl.when(n > 0)
    def _():
        acc_ref[...] = acc_ref[...] + part

    @pl.when(n == N - 1)
    def _():
        h = jax.nn.relu(acc_ref[...])
        h = jax.nn.relu(_dot(h, d2w_ref[...]) + d2b_ref[...])
        t = jnp.tanh(_dot(h, dtw_ref[...]) + dtb_ref[...])
        zm = jax.nn.relu(_dot(t, zmw_ref[...]) + zmb_ref[...])
        zv = jax.nn.relu(_dot(t, zvw_ref[...]) + zvb_ref[...])
        zm_ref[...] = zm
        zv_ref[...] = zv
        z_ref[...] = zm + jnp.exp(0.5 * zv) * eps_ref[...]


def kernel(x, a, W1, b1, W2, b2, W3, b3, W4, b4,
           D1W, D1b, D2W, D2b, DtW, Dtb, ZmW, Zmb, ZvW, Zvb):
    f32 = jnp.float32
    row = lambda v: v.reshape(1, -1)

    full = lambda s: pl.BlockSpec(s, lambda i: (0,) * len(s))
    batched = lambda s: pl.BlockSpec(s, lambda i: (i,) + (0,) * (len(s) - 1))

    h4 = pl.pallas_call(
        _conv_kernel,
        grid=(B // BB,),
        in_specs=[
            batched((BB, N, F)),
            batched((BB, N, N)),
            full((F, H)), full((1, H)),
            full((H, H)), full((1, H)),
            full((H, H)), full((1, H)),
            full((H, H2)), full((1, H2)),
        ],
        out_specs=pl.BlockSpec((N, BB, H2), lambda i: (0, i, 0)),
        out_shape=jax.ShapeDtypeStruct((N, B, H2), f32),
    )(x, a, W1, row(b1), W2, row(b2), W3, row(b3), W4, row(b4))

    eps = jnp.asarray(_EPS)

    full0 = lambda s: pl.BlockSpec(s, lambda n: (0,) * len(s))
    out_shapes = [jax.ShapeDtypeStruct((B, L), f32)] * 3
    zm, zv, z = pl.pallas_call(
        _dense_kernel,
        grid=(N,),
        in_specs=[
            pl.BlockSpec((1, B, H2), lambda n: (n, 0, 0)),
            pl.BlockSpec((H2, H), lambda n: (n, 0)),
            full0((1, H)),
            full0((H, H)), full0((1, H)),
            full0((H, 4 * L)), full0((1, 4 * L)),
            full0((4 * L, L)), full0((1, L)),
            full0((4 * L, L)), full0((1, L)),
            full0((B, L)),
        ],
        out_specs=[full0((B, L))] * 3,
        out_shape=out_shapes,
        scratch_shapes=[pltpu.VMEM((B, H), f32)],
    )(h4, D1W, row(D1b), D2W, row(D2b), DtW, row(Dtb),
      ZmW, row(Zmb), ZvW, row(Zvb), eps)
    return (zm, zv, z)


# speedup vs baseline: 2.0003x; 2.0003x over previous
import jax
import jax.numpy as jnp
from jax.experimental import pallas as pl
from jax.experimental.pallas import tpu as pltpu

B, N, F = 256, 128, 64
H = 256
H2 = 128
L = 64

BB = 32
GRID = B // BB


def _dot(a_, b_):
    return jnp.dot(a_, b_, preferred_element_type=jnp.float32)


def _fused_kernel(x_ref, a_ref, w1_ref, b1_ref, w2_ref, b2_ref,
                  w3_ref, b3_ref, w4_ref, b4_ref,
                  d1w_ref, d1b_ref, d2w_ref, d2b_ref,
                  dtw_ref, dtb_ref, zmw_ref, zmb_ref, zvw_ref, zvb_ref,
                  eps_ref, zm_ref, zv_ref, z_ref, h4_ref):
    i = pl.program_id(0)
    h = x_ref[...].reshape(BB * N, F)

    def gcn(h2d, w_ref, b_ref):
        m = _dot(h2d, w_ref[...])
        b = b_ref[...]
        outs = []
        for g in range(BB):
            ag = a_ref[g]
            mg = m[g * N:(g + 1) * N]
            outs.append(jax.nn.relu(_dot(ag, mg) + b))
        return jnp.concatenate(outs, axis=0)

    h = gcn(h, w1_ref, b1_ref)
    h = gcn(h, w2_ref, b2_ref)
    h = gcn(h, w3_ref, b3_ref)
    m = _dot(h, w4_ref[...])
    b4 = b4_ref[...]
    for g in range(BB):
        out_g = jax.nn.relu(_dot(a_ref[g], m[g * N:(g + 1) * N]) + b4)
        h4_ref[pl.ds(i * BB + g, 1)] = out_g[None]

    @pl.when(i == GRID - 1)
    def _():
        acc = jnp.zeros((B, H), dtype=jnp.float32)
        for n in range(N):
            acc = acc + _dot(h4_ref[:, n, :], d1w_ref[n * H2:(n + 1) * H2])
        h1 = jax.nn.relu(acc + d1b_ref[...])
        h2 = jax.nn.relu(_dot(h1, d2w_ref[...]) + d2b_ref[...])
        t = jnp.tanh(_dot(h2, dtw_ref[...]) + dtb_ref[...])
        zm = jax.nn.relu(_dot(t, zmw_ref[...]) + zmb_ref[...])
        zv = jax.nn.relu(_dot(t, zvw_ref[...]) + zvb_ref[...])
        zm_ref[...] = zm
        zv_ref[...] = zv
        z_ref[...] = zm + jnp.exp(0.5 * zv) * eps_ref[...]


def kernel(x, a, W1, b1, W2, b2, W3, b3, W4, b4,
           D1W, D1b, D2W, D2b, DtW, Dtb, ZmW, Zmb, ZvW, Zvb):
    f32 = jnp.float32
    row = lambda v: v.reshape(1, -1)

    full = lambda s: pl.BlockSpec(s, lambda i: (0,) * len(s))
    batched = lambda s: pl.BlockSpec(s, lambda i: (i,) + (0,) * (len(s) - 1))

    eps = jax.random.normal(jax.random.key(1), (B, L), dtype=f32)
    out_shapes = [jax.ShapeDtypeStruct((B, L), f32)] * 3
    zm, zv, z = pl.pallas_call(
        _fused_kernel,
        grid=(GRID,),
        in_specs=[
            batched((BB, N, F)),
            batched((BB, N, N)),
            full((F, H)), full((1, H)),
            full((H, H)), full((1, H)),
            full((H, H)), full((1, H)),
            full((H, H2)), full((1, H2)),
            full((N * H2, H)), full((1, H)),
            full((H, H)), full((1, H)),
            full((H, 4 * L)), full((1, 4 * L)),
            full((4 * L, L)), full((1, L)),
            full((4 * L, L)), full((1, L)),
            full((B, L)),
        ],
        out_specs=[full((B, L))] * 3,
        out_shape=out_shapes,
        scratch_shapes=[pltpu.VMEM((B, N, H2), f32)],
    )(x, a, W1, row(b1), W2, row(b2), W3, row(b3), W4, row(b4),
      D1W, row(D1b), D2W, row(D2b), DtW, row(Dtb),
      ZmW, row(Zmb), ZvW, row(Zvb), eps)
    return (zm, zv, z)


# speedup vs baseline: 2.0672x; 1.0335x over previous
import jax
import jax.numpy as jnp
from jax.experimental import pallas as pl
from jax.experimental.pallas import tpu as pltpu

B, N, F = 256, 128, 64
H = 256
H2 = 128
L = 64

BB = 32
GRID = B // BB


def _dot(a_, b_):
    return jnp.dot(a_, b_, preferred_element_type=jnp.float32)


def _fused_kernel(x_ref, a_ref, w1_ref, b1_ref, w2_ref, b2_ref,
                  w3_ref, b3_ref, w4_ref, b4_ref,
                  d1w_ref, d1b_ref, d2w_ref, d2b_ref,
                  dtw_ref, dtb_ref, zmw_ref, zmb_ref, zvw_ref, zvb_ref,
                  eps_ref, zm_ref, zv_ref, z_ref, h4_ref):
    i = pl.program_id(0)
    h = x_ref[...].reshape(BB * N, F)

    def gcn(h2d, w_ref, b_ref):
        m = _dot(h2d, w_ref[...])
        b = b_ref[...]
        outs = []
        for g in range(BB):
            ag = a_ref[g]
            mg = m[g * N:(g + 1) * N]
            outs.append(jax.nn.relu(_dot(ag, mg) + b))
        return jnp.concatenate(outs, axis=0)

    h = gcn(h, w1_ref, b1_ref)
    h = gcn(h, w2_ref, b2_ref)
    h = gcn(h, w3_ref, b3_ref)
    m = _dot(h, w4_ref[...])
    b4 = b4_ref[...]
    outs = []
    for g in range(BB):
        outs.append(jax.nn.relu(_dot(a_ref[g], m[g * N:(g + 1) * N]) + b4))
    tr = jnp.transpose(jnp.stack(outs), (1, 0, 2))
    h4_ref[:, pl.ds(i * BB, BB), :] = tr

    @pl.when(i == GRID - 1)
    def _():
        acc = jnp.zeros((B, H), dtype=jnp.float32)
        for n in range(N):
            acc = acc + _dot(h4_ref[n], d1w_ref[n * H2:(n + 1) * H2])
        h1 = jax.nn.relu(acc + d1b_ref[...])
        h2 = jax.nn.relu(_dot(h1, d2w_ref[...]) + d2b_ref[...])
        t = jnp.tanh(_dot(h2, dtw_ref[...]) + dtb_ref[...])
        zm = jax.nn.relu(_dot(t, zmw_ref[...]) + zmb_ref[...])
        zv = jax.nn.relu(_dot(t, zvw_ref[...]) + zvb_ref[...])
        zm_ref[...] = zm
        zv_ref[...] = zv
        z_ref[...] = zm + jnp.exp(0.5 * zv) * eps_ref[...]


def kernel(x, a, W1, b1, W2, b2, W3, b3, W4, b4,
           D1W, D1b, D2W, D2b, DtW, Dtb, ZmW, Zmb, ZvW, Zvb):
    f32 = jnp.float32
    row = lambda v: v.reshape(1, -1)

    full = lambda s: pl.BlockSpec(s, lambda i: (0,) * len(s))
    batched = lambda s: pl.BlockSpec(s, lambda i: (i,) + (0,) * (len(s) - 1))

    eps = jax.random.normal(jax.random.key(1), (B, L), dtype=f32)
    out_shapes = [jax.ShapeDtypeStruct((B, L), f32)] * 3
    zm, zv, z = pl.pallas_call(
        _fused_kernel,
        grid=(GRID,),
        in_specs=[
            batched((BB, N, F)),
            batched((BB, N, N)),
            full((F, H)), full((1, H)),
            full((H, H)), full((1, H)),
            full((H, H)), full((1, H)),
            full((H, H2)), full((1, H2)),
            full((N * H2, H)), full((1, H)),
            full((H, H)), full((1, H)),
            full((H, 4 * L)), full((1, 4 * L)),
            full((4 * L, L)), full((1, L)),
            full((4 * L, L)), full((1, L)),
            full((B, L)),
        ],
        out_specs=[full((B, L))] * 3,
        out_shape=out_shapes,
        scratch_shapes=[pltpu.VMEM((N, B, H2), f32)],
    )(x, a, W1, row(b1), W2, row(b2), W3, row(b3), W4, row(b4),
      D1W, row(D1b), D2W, row(D2b), DtW, row(Dtb),
      ZmW, row(Zmb), ZvW, row(Zvb), eps)
    return (zm, zv, z)


# speedup vs baseline: 2.1243x; 1.0276x over previous
import jax
import jax.numpy as jnp
from jax.experimental import pallas as pl
from jax.experimental.pallas import tpu as pltpu

B, N, F = 256, 128, 64
H = 256
H2 = 128
L = 64

BB = 32
GRID = B // BB
SLAB = (N * H2) // GRID


def _dot(a_, b_):
    return jnp.dot(a_, b_, preferred_element_type=jnp.float32)


def _fused_kernel(x_ref, a_ref, w1_ref, b1_ref, w2_ref, b2_ref,
                  w3_ref, b3_ref, w4_ref, b4_ref,
                  d1w_ref, d1b_ref, d2w_ref, d2b_ref,
                  dtw_ref, dtb_ref, zmw_ref, zmb_ref, zvw_ref, zvb_ref,
                  eps_ref, zm_ref, zv_ref, z_ref, h4_ref, d1w_vmem):
    i = pl.program_id(0)
    d1w_vmem[pl.ds(i * SLAB, SLAB)] = d1w_ref[0]
    h = x_ref[...].reshape(BB * N, F)

    def gcn(h2d, w_ref, b_ref):
        m = _dot(h2d, w_ref[...])
        b = b_ref[...]
        outs = []
        for g in range(BB):
            ag = a_ref[g]
            mg = m[g * N:(g + 1) * N]
            outs.append(jax.nn.relu(_dot(ag, mg) + b))
        return jnp.concatenate(outs, axis=0)

    h = gcn(h, w1_ref, b1_ref)
    h = gcn(h, w2_ref, b2_ref)
    h = gcn(h, w3_ref, b3_ref)
    m = _dot(h, w4_ref[...])
    b4 = b4_ref[...]
    outs = []
    for g in range(BB):
        outs.append(jax.nn.relu(_dot(a_ref[g], m[g * N:(g + 1) * N]) + b4))
    tr = jnp.transpose(jnp.stack(outs), (1, 0, 2))
    h4_ref[:, pl.ds(i * BB, BB), :] = tr

    @pl.when(i == GRID - 1)
    def _():
        acc = jnp.zeros((B, H), dtype=jnp.float32)
        for n in range(N):
            acc = acc + _dot(h4_ref[n], d1w_vmem[n * H2:(n + 1) * H2])
        h1 = jax.nn.relu(acc + d1b_ref[...])
        h2 = jax.nn.relu(_dot(h1, d2w_ref[...]) + d2b_ref[...])
        t = jnp.tanh(_dot(h2, dtw_ref[...]) + dtb_ref[...])
        zm = jax.nn.relu(_dot(t, zmw_ref[...]) + zmb_ref[...])
        zv = jax.nn.relu(_dot(t, zvw_ref[...]) + zvb_ref[...])
        zm_ref[...] = zm
        zv_ref[...] = zv
        z_ref[...] = zm + jnp.exp(0.5 * zv) * eps_ref[...]


def kernel(x, a, W1, b1, W2, b2, W3, b3, W4, b4,
           D1W, D1b, D2W, D2b, DtW, Dtb, ZmW, Zmb, ZvW, Zvb):
    f32 = jnp.float32
    row = lambda v: v.reshape(1, -1)

    full = lambda s: pl.BlockSpec(s, lambda i: (0,) * len(s))
    batched = lambda s: pl.BlockSpec(s, lambda i: (i,) + (0,) * (len(s) - 1))

    eps = jax.random.normal(jax.random.key(1), (B, L), dtype=f32)
    out_shapes = [jax.ShapeDtypeStruct((B, L), f32)] * 3
    zm, zv, z = pl.pallas_call(
        _fused_kernel,
        grid=(GRID,),
        in_specs=[
            batched((BB, N, F)),
            batched((BB, N, N)),
            full((F, H)), full((1, H)),
            full((H, H)), full((1, H)),
            full((H, H)), full((1, H)),
            full((H, H2)), full((1, H2)),
            pl.BlockSpec((1, SLAB, H), lambda i: (i, 0, 0)), full((1, H)),
            full((H, H)), full((1, H)),
            full((H, 4 * L)), full((1, 4 * L)),
            full((4 * L, L)), full((1, L)),
            full((4 * L, L)), full((1, L)),
            full((B, L)),
        ],
        out_specs=[full((B, L))] * 3,
        out_shape=out_shapes,
        scratch_shapes=[pltpu.VMEM((N, B, H2), f32),
                        pltpu.VMEM((N * H2, H), f32)],
    )(x, a, W1, row(b1), W2, row(b2), W3, row(b3), W4, row(b4),
      D1W.reshape(GRID, SLAB, H), row(D1b), D2W, row(D2b), DtW, row(Dtb),
      ZmW, row(Zmb), ZvW, row(Zvb), eps)
    return (zm, zv, z)


# speedup vs baseline: 2.1244x; 1.0001x over previous
import jax
import jax.numpy as jnp
from jax.experimental import pallas as pl
from jax.experimental.pallas import tpu as pltpu

B, N, F = 256, 128, 64
H = 256
H2 = 128
L = 64

BB = 32
GRID = B // BB
SLAB = (N * H2) // GRID


def _dot(a_, b_):
    return jnp.dot(a_, b_, preferred_element_type=jnp.float32)


def _fused_kernel(x_ref, a_ref, w1_ref, b1_ref, w2_ref, b2_ref,
                  w3_ref, b3_ref, w4_ref, b4_ref,
                  d1w_ref, d1b_ref, d2w_ref, d2b_ref,
                  dtw_ref, dtb_ref, zmw_ref, zmb_ref, zvw_ref, zvb_ref,
                  eps_ref, zm_ref, zv_ref, z_ref, h4_ref, d1w_vmem):
    i = pl.program_id(0)
    d1w_vmem[pl.ds(i * SLAB, SLAB)] = d1w_ref[...]
    h = x_ref[...].reshape(BB * N, F)

    def gcn(h2d, w_ref, b_ref):
        m = _dot(h2d, w_ref[...])
        b = b_ref[...]
        outs = []
        for g in range(BB):
            ag = a_ref[g]
            mg = m[g * N:(g + 1) * N]
            outs.append(jax.nn.relu(_dot(ag, mg) + b))
        return jnp.concatenate(outs, axis=0)

    h = gcn(h, w1_ref, b1_ref)
    h = gcn(h, w2_ref, b2_ref)
    h = gcn(h, w3_ref, b3_ref)
    m = _dot(h, w4_ref[...])
    b4 = b4_ref[...]
    outs = []
    for g in range(BB):
        outs.append(jax.nn.relu(_dot(a_ref[g], m[g * N:(g + 1) * N]) + b4))
    tr = jnp.transpose(jnp.stack(outs), (1, 0, 2))
    h4_ref[:, pl.ds(i * BB, BB), :] = tr

    @pl.when(i == GRID - 1)
    def _():
        acc = jnp.zeros((B, H), dtype=jnp.float32)
        for n in range(N):
            acc = acc + _dot(h4_ref[n], d1w_vmem[n * H2:(n + 1) * H2])
        h1 = jax.nn.relu(acc + d1b_ref[...])
        h2 = jax.nn.relu(_dot(h1, d2w_ref[...]) + d2b_ref[...])
        t = jnp.tanh(_dot(h2, dtw_ref[...]) + dtb_ref[...])
        zm = jax.nn.relu(_dot(t, zmw_ref[...]) + zmb_ref[...])
        zv = jax.nn.relu(_dot(t, zvw_ref[...]) + zvb_ref[...])
        zm_ref[...] = zm
        zv_ref[...] = zv
        z_ref[...] = zm + jnp.exp(0.5 * zv) * eps_ref[...]


def kernel(x, a, W1, b1, W2, b2, W3, b3, W4, b4,
           D1W, D1b, D2W, D2b, DtW, Dtb, ZmW, Zmb, ZvW, Zvb):
    f32 = jnp.float32
    row = lambda v: v.reshape(1, -1)

    full = lambda s: pl.BlockSpec(s, lambda i: (0,) * len(s))
    batched = lambda s: pl.BlockSpec(s, lambda i: (i,) + (0,) * (len(s) - 1))

    eps = jax.random.normal(jax.random.key(1), (B, L), dtype=f32)
    out_shapes = [jax.ShapeDtypeStruct((B, L), f32)] * 3
    zm, zv, z = pl.pallas_call(
        _fused_kernel,
        grid=(GRID,),
        in_specs=[
            batched((BB, N, F)),
            batched((BB, N, N)),
            full((F, H)), full((1, H)),
            full((H, H)), full((1, H)),
            full((H, H)), full((1, H)),
            full((H, H2)), full((1, H2)),
            pl.BlockSpec((SLAB, H), lambda i: (i, 0)), full((1, H)),
            full((H, H)), full((1, H)),
            full((H, 4 * L)), full((1, 4 * L)),
            full((4 * L, L)), full((1, L)),
            full((4 * L, L)), full((1, L)),
            full((B, L)),
        ],
        out_specs=[full((B, L))] * 3,
        out_shape=out_shapes,
        scratch_shapes=[pltpu.VMEM((N, B, H2), f32),
                        pltpu.VMEM((N * H2, H), f32)],
    )(x, a, W1, row(b1), W2, row(b2), W3, row(b3), W4, row(b4),
      D1W, row(D1b), D2W, row(D2b), DtW, row(Dtb),
      ZmW, row(Zmb), ZvW, row(Zvb), eps)
    return (zm, zv, z)
